# baseline (speedup 1.0000x reference)
import functools

import jax
import jax.numpy as jnp
from jax import lax
from jax.experimental import pallas as pl
from jax.experimental.pallas import tpu as pltpu
from jax.experimental.pallas import tpu_sc as plsc

_NS = 50000
_NW = 25000
_DIN = 128
_H1 = 4
_C1 = 64
_C2 = 128

_NSPAD = 50176
_NWPAD = 25088

_NCORES = 2
_NSUB = 16
_NWORKERS = _NCORES * _NSUB
_CR = 4
_ECHUNK = _NWORKERS * _CR * 128

_EPS = 1e-16


def _round_up(x, m):
    return (x + m - 1) // m * m


def _dense_fields(x, wt, bias, gs, n):
    R = 512
    ng = len(gs)

    def body(*refs):
        x_ref, wt_ref, b_ref = refs[0], refs[1], refs[2]
        g_refs = refs[3:3 + ng]
        o_refs = refs[3 + ng:]
        h = jnp.dot(x_ref[...], wt_ref[...], preferred_element_type=jnp.float32)
        h = jnp.maximum(h + b_ref[...], 0.0)
        for g_ref, o_ref in zip(g_refs, o_refs):
            o_ref[...] = jnp.dot(h, g_ref[...], preferred_element_type=jnp.float32)

    in_specs = [
        pl.BlockSpec((R, _DIN), lambda i: (i, 0)),
        pl.BlockSpec((_DIN, _DIN), lambda i: (0, 0)),
        pl.BlockSpec((1, _DIN), lambda i: (0, 0)),
    ] + [pl.BlockSpec((_DIN, g.shape[1]), lambda i: (0, 0)) for g in gs]
    out_specs = [pl.BlockSpec((R, g.shape[1]), lambda i: (i, 0)) for g in gs]
    out_shape = [jax.ShapeDtypeStruct((n, g.shape[1]), jnp.float32) for g in gs]
    return pl.pallas_call(
        body,
        grid=(n // R,),
        in_specs=in_specs,
        out_specs=out_specs,
        out_shape=out_shape,
    )(x, wt, bias, *gs)


def _make_edge_gat(e_pad, n_dst_pad, cs, cd, h_heads, j_proj,
                   src_a_col, src_p_col, dst_a_col):
    cacc = h_heads * (1 + j_proj)
    n_rows = e_pad // 128
    rows_per_worker = n_rows // _NWORKERS
    n_chunks = rows_per_worker // _CR
    rows_per_sub = n_dst_pad // _NSUB
    mesh = plsc.VectorSubcoreMesh(core_axis_name="c", subcore_axis_name="s",
                                  num_cores=_NCORES, num_subcores=_NSUB)

    @functools.partial(
        pl.kernel,
        out_type=jax.ShapeDtypeStruct((_NCORES, n_dst_pad, cacc), jnp.float32),
        mesh=mesh,
        compiler_params=pltpu.CompilerParams(needs_layout_passes=False,
                                             use_tc_tiling_on_sc=False),
        scratch_types=[
            pltpu.VMEM((_CR * 128,), jnp.int32),
            pltpu.VMEM((_CR * 128,), jnp.int32),
            pltpu.VMEM((_CR * 128, cs), jnp.float32),
            pltpu.VMEM((_CR * 128, cd), jnp.float32),
            pltpu.VMEM((_CR * 128, cacc), jnp.float32),
            pltpu.VMEM_SHARED((n_dst_pad, cacc), jnp.float32),
            pltpu.SemaphoreType.DMA,
        ],
    )
    def k(src_tab, dst_tab, src_idx, dst_idx, zeros_hbm, out,
          sidx, didx, sv, dv, ov, acc, sem):
        cid = lax.axis_index("c")
        sid = lax.axis_index("s")
        wid = sid * _NCORES + cid
        pltpu.sync_copy(zeros_hbm, acc.at[pl.ds(sid * rows_per_sub, rows_per_sub)])
        plsc.subcore_barrier()

        row0 = wid * rows_per_worker
        iota = lax.iota(jnp.int32, 16)

        def chunk(it, carry):
            rb = (row0 + it * _CR) * 128
            pltpu.sync_copy(src_idx.at[pl.ds(rb, _CR * 128)], sidx)
            pltpu.sync_copy(dst_idx.at[pl.ds(rb, _CR * 128)], didx)
            cps = [pltpu.async_copy(src_tab.at[sidx], sv, sem),
                   pltpu.async_copy(dst_tab.at[didx], dv, sem)]
            for cp in cps:
                cp.wait()
            for r in range(_CR):
                for g in range(8):
                    rows = r * 128 + g * 16 + iota
                    for h in range(h_heads):
                        a = plsc.load_gather(
                            sv, [rows,
                                 jnp.full((16,), src_a_col + h, jnp.int32)])
                        b = plsc.load_gather(
                            dv, [rows,
                                 jnp.full((16,), dst_a_col + h, jnp.int32)])
                        s = a + b
                        e = jnp.where(s >= 0, s, s * jnp.float32(0.2))
                        w = jnp.exp(e)
                        plsc.store_scatter(
                            ov, [rows, jnp.full((16,), h, jnp.int32)], w)
                        for j in range(j_proj):
                            p = plsc.load_gather(
                                sv, [rows,
                                     jnp.full((16,), src_p_col + h * j_proj + j,
                                              jnp.int32)])
                            plsc.store_scatter(
                                ov, [rows,
                                     jnp.full((16,), h_heads + h * j_proj + j,
                                              jnp.int32)], w * p)
            pltpu.sync_copy(ov, acc.at[didx], add=True)
            return carry

        lax.fori_loop(0, n_chunks, chunk, 0)
        plsc.subcore_barrier()
        pltpu.sync_copy(acc.at[pl.ds(sid * rows_per_sub, rows_per_sub)],
                        out.at[cid].at[pl.ds(sid * rows_per_sub, rows_per_sub)])

    return k


def _pad_edges(src, dst, pad_dst):
    e = src.shape[0]
    e_pad = _round_up(e, _ECHUNK)
    src_p = jnp.pad(src, (0, e_pad - e))
    dst_p = jnp.pad(dst, (0, e_pad - e), constant_values=pad_dst)
    return src_p, dst_p, e_pad


def _combine_l1(accs, bias, h_heads, j_proj, sel_b, sel_s):
    n = accs[0].shape[1]
    c = accs[0].shape[2]
    R = 512
    na = len(accs)

    def body(*refs):
        a_refs = refs[:na]
        b_ref, selb_ref, sels_ref, o_ref = refs[na:]
        v = None
        for a_ref in a_refs:
            a = a_ref[0] + a_ref[1]
            inv = 1.0 / (a + _EPS)
            t = a * jnp.dot(inv, selb_ref[...],
                            preferred_element_type=jnp.float32)
            vj = jnp.dot(t, sels_ref[...], preferred_element_type=jnp.float32)
            v = vj if v is None else v + vj
        o_ref[...] = v + b_ref[...]

    in_specs = (
        [pl.BlockSpec((2, R, c), lambda i: (0, i, 0)) for _ in accs]
        + [pl.BlockSpec((1, j_proj), lambda i: (0, 0)),
           pl.BlockSpec((c, c), lambda i: (0, 0)),
           pl.BlockSpec((c, j_proj), lambda i: (0, 0))]
    )
    return pl.pallas_call(
        body,
        grid=(n // R,),
        in_specs=in_specs,
        out_specs=pl.BlockSpec((R, j_proj), lambda i: (i, 0)),
        out_shape=jax.ShapeDtypeStruct((n, j_proj), jnp.float32),
    )(*accs, bias, sel_b, sel_s)


def _combine_final(acc_a, acc_b, c0):
    n = acc_a.shape[1]
    R = 1024

    def body(a_ref, b_ref, c_ref, o_ref):
        a = a_ref[0] + a_ref[1]
        b = b_ref[0] + b_ref[1]
        r = a[:, 1] / (a[:, 0] + _EPS) + b[:, 1] / (b[:, 0] + _EPS)
        o_ref[...] = (r + c_ref[0]).reshape(R // 128, 128)

    return pl.pallas_call(
        body,
        grid=(n // R,),
        in_specs=[
            pl.BlockSpec((2, R, 2), lambda i: (0, i, 0)),
            pl.BlockSpec((2, R, 2), lambda i: (0, i, 0)),
            pl.BlockSpec(memory_space=pltpu.SMEM),
        ],
        out_specs=pl.BlockSpec((R // 128, 128), lambda i: (i, 0)),
        out_shape=jax.ShapeDtypeStruct((n // 128, 128), jnp.float32),
    )(acc_a, acc_b, c0)


def kernel(sentence_feat, word_feat, W_sent, b_sent, W_word, b_word,
           W_ss1, a_src_ss1, a_dst_ss1, b_ss1,
           W_sw1, a_src_sw1, a_dst_sw1, b_sw1,
           W_ws1, a_src_ws1, a_dst_ws1, b_ws1,
           W_ss2, a_src_ss2, a_dst_ss2, b_ss2,
           W_sw2, a_src_sw2, a_dst_sw2, b_sw2,
           W_ws2, a_src_ws2, a_dst_ws2, b_ws2,
           ei_ss_src, ei_ss_dst, ei_sw_src, ei_sw_dst, ei_ws_src, ei_ws_dst):
    f32 = jnp.float32

    V_sent = jnp.stack([W_ss2 @ a_src_ss2[0], W_ss2 @ a_dst_ss2[0],
                        W_ss2.mean(axis=1), W_ws2 @ a_dst_ws2[0]], axis=1)
    V_word = jnp.stack([W_ws2 @ a_src_ws2[0], W_ws2.mean(axis=1)], axis=1)

    def fold(W1, a_s, a_d, V):
        W1r = W1.reshape(_DIN, _H1, _C1)
        A_s = jnp.einsum('dhc,hc->dh', W1r, a_s)
        A_d = jnp.einsum('dhc,hc->dh', W1r, a_d)
        M = jnp.einsum('dhc,hcj->dhj', W1r,
                       V.reshape(_H1, _C1, V.shape[1]))
        return A_s, A_d, M.reshape(_DIN, _H1 * V.shape[1])

    As_ss, Ad_ss, M_ss = fold(W_ss1, a_src_ss1, a_dst_ss1, V_sent)
    As_sw, Ad_sw, M_sw = fold(W_sw1, a_src_sw1, a_dst_sw1, V_word)
    As_ws, Ad_ws, M_ws = fold(W_ws1, a_src_ws1, a_dst_ws1, V_sent)

    G_src_ss = jnp.concatenate([As_ss, M_ss], axis=1)
    G_src_sw = jnp.concatenate([As_sw, M_sw], axis=1)
    G_src_ws = jnp.concatenate([As_ws, M_ws], axis=1)

    bias_sent = ((b_ss1 + b_ws1) @ V_sent).reshape(1, 4)
    bias_word = (b_sw1 @ V_word).reshape(1, 2)
    c0 = (b_ss2.mean() + b_ws2.mean()).reshape(1)

    bs = b_sent.reshape(1, _DIN)
    bw = b_word.reshape(1, _DIN)

    T_src_ss, T_dst_ss, T_src_sw, T_dst_ws = _dense_fields(
        sentence_feat, W_sent.T, bs, [G_src_ss, Ad_ss, G_src_sw, Ad_ws],
        _NSPAD)
    T_dst_sw, T_src_ws = _dense_fields(word_feat, W_word.T, bw,
                                       [Ad_sw, G_src_ws], _NWPAD)

    ss_s, ss_d, e_ss = _pad_edges(ei_ss_src, ei_ss_dst, _NS)
    sw_s, sw_d, e_sw = _pad_edges(ei_sw_src, ei_sw_dst, _NW)
    ws_s, ws_d, e_ws = _pad_edges(ei_ws_src, ei_ws_dst, _NS)

    z_ns20 = jnp.zeros((_NSPAD // _NSUB, 20), f32)
    z_nw12 = jnp.zeros((_NWPAD // _NSUB, 12), f32)
    z_ns2 = jnp.zeros((_NSPAD // _NSUB, 2), f32)

    acc_ss = _make_edge_gat(e_ss, _NSPAD, 20, 4, _H1, 4, 0, 4, 0)(
        T_src_ss, T_dst_ss, ss_s, ss_d, z_ns20)
    acc_sw = _make_edge_gat(e_sw, _NWPAD, 12, 4, _H1, 2, 0, 4, 0)(
        T_src_sw, T_dst_sw, sw_s, sw_d, z_nw12)
    acc_ws = _make_edge_gat(e_ws, _NSPAD, 20, 4, _H1, 4, 0, 4, 0)(
        T_src_ws, T_dst_ws, ws_s, ws_d, z_ns20)

    def sel_mats(h_heads, j_proj):
        c = h_heads * (1 + j_proj)
        sb = jnp.zeros((c, c), f32)
        ss = jnp.zeros((c, j_proj), f32)
        for h in range(h_heads):
            for j in range(j_proj):
                sb = sb.at[h, h_heads + h * j_proj + j].set(1.0)
                ss = ss.at[h_heads + h * j_proj + j, j].set(1.0)
        return sb, ss

    sb4, ss4 = sel_mats(_H1, 4)
    sb2, ss2m = sel_mats(_H1, 2)
    proj_sent = _combine_l1([acc_ss, acc_ws], bias_sent, _H1, 4, sb4, ss4)
    proj_word = _combine_l1([acc_sw], bias_word, _H1, 2, sb2, ss2m)

    acc_ss2 = _make_edge_gat(e_ss, _NSPAD, 4, 4, 1, 1, 0, 2, 1)(
        proj_sent, proj_sent, ss_s, ss_d, z_ns2)
    acc_ws2 = _make_edge_gat(e_ws, _NSPAD, 2, 4, 1, 1, 0, 1, 3)(
        proj_word, proj_sent, ws_s, ws_d, z_ns2)

    out = _combine_final(acc_ss2, acc_ws2, c0)
    return out.reshape(_NSPAD)[:_NS]


# speedup vs baseline: 1.1494x; 1.1494x over previous
import functools

import jax
import jax.numpy as jnp
from jax import lax
from jax.experimental import pallas as pl
from jax.experimental.pallas import tpu as pltpu
from jax.experimental.pallas import tpu_sc as plsc

_NS = 50000
_NW = 25000
_DIN = 128
_H1 = 4
_C1 = 64
_C2 = 128

_NSPAD = 50176
_NWPAD = 25088

_NCORES = 2
_NSUB = 16
_NWORKERS = _NCORES * _NSUB
_CR = 2
_ECHUNK = _NWORKERS * _CR * 128

_EPS = 1e-16


def _round_up(x, m):
    return (x + m - 1) // m * m


def _dense_fields(x, wt, bias, gs, n):
    R = 512
    ng = len(gs)

    def body(*refs):
        x_ref, wt_ref, b_ref = refs[0], refs[1], refs[2]
        g_refs = refs[3:3 + ng]
        o_refs = refs[3 + ng:]
        h = jnp.dot(x_ref[...], wt_ref[...], preferred_element_type=jnp.float32)
        h = jnp.maximum(h + b_ref[...], 0.0)
        for g_ref, o_ref in zip(g_refs, o_refs):
            o_ref[...] = jnp.dot(h, g_ref[...], preferred_element_type=jnp.float32)

    in_specs = [
        pl.BlockSpec((R, _DIN), lambda i: (i, 0)),
        pl.BlockSpec((_DIN, _DIN), lambda i: (0, 0)),
        pl.BlockSpec((1, _DIN), lambda i: (0, 0)),
    ] + [pl.BlockSpec((_DIN, g.shape[1]), lambda i: (0, 0)) for g in gs]
    out_specs = [pl.BlockSpec((R, g.shape[1]), lambda i: (i, 0)) for g in gs]
    out_shape = [jax.ShapeDtypeStruct((n, g.shape[1]), jnp.float32) for g in gs]
    return pl.pallas_call(
        body,
        grid=(n // R,),
        in_specs=in_specs,
        out_specs=out_specs,
        out_shape=out_shape,
    )(x, wt, bias, *gs)


def _make_edge_gat(e_pad, n_dst_pad, cs, cd, h_heads, j_proj,
                   src_a_col, src_p_col, dst_a_col):
    cacc = h_heads * (1 + j_proj)
    n_rows = e_pad // 128
    rows_per_worker = n_rows // _NWORKERS
    n_chunks = rows_per_worker // _CR
    rows_per_sub = n_dst_pad // _NSUB
    mesh = plsc.VectorSubcoreMesh(core_axis_name="c", subcore_axis_name="s",
                                  num_cores=_NCORES, num_subcores=_NSUB)

    @functools.partial(
        pl.kernel,
        out_type=jax.ShapeDtypeStruct((_NCORES, n_dst_pad, cacc), jnp.float32),
        mesh=mesh,
        compiler_params=pltpu.CompilerParams(needs_layout_passes=False,
                                             use_tc_tiling_on_sc=False),
        scratch_types=[
            pltpu.VMEM((_CR * 128,), jnp.int32),
            pltpu.VMEM((_CR * 128,), jnp.int32),
            pltpu.VMEM((_CR * 128, cs), jnp.float32),
            pltpu.VMEM((_CR * 128, cd), jnp.float32),
            pltpu.VMEM((_CR * 128, cacc), jnp.float32),
            pltpu.VMEM_SHARED((n_dst_pad, cacc), jnp.float32),
            pltpu.SemaphoreType.DMA,
        ],
    )
    def k(src_tab, dst_tab, src_idx, dst_idx, zeros_hbm, out,
          sidx, didx, sv, dv, ov, acc, sem):
        cid = lax.axis_index("c")
        sid = lax.axis_index("s")
        wid = sid * _NCORES + cid
        pltpu.sync_copy(zeros_hbm, acc.at[pl.ds(sid * rows_per_sub, rows_per_sub)])
        plsc.subcore_barrier()

        row0 = wid * rows_per_worker
        iota = lax.iota(jnp.int32, 16)

        def chunk(it, carry):
            rb = (row0 + it * _CR) * 128
            pltpu.sync_copy(src_idx.at[pl.ds(rb, _CR * 128)], sidx)
            pltpu.sync_copy(dst_idx.at[pl.ds(rb, _CR * 128)], didx)
            cps = [pltpu.async_copy(src_tab.at[sidx], sv, sem),
                   pltpu.async_copy(dst_tab.at[didx], dv, sem)]
            for cp in cps:
                cp.wait()
            for r in range(_CR):
                for g in range(8):
                    rows = r * 128 + g * 16 + iota
                    for h in range(h_heads):
                        a = plsc.load_gather(
                            sv, [rows,
                                 jnp.full((16,), src_a_col + h, jnp.int32)])
                        b = plsc.load_gather(
                            dv, [rows,
                                 jnp.full((16,), dst_a_col + h, jnp.int32)])
                        s = a + b
                        e = jnp.where(s >= 0, s, s * jnp.float32(0.2))
                        w = jnp.exp(e)
                        plsc.store_scatter(
                            ov, [rows, jnp.full((16,), h, jnp.int32)], w)
                        for j in range(j_proj):
                            p = plsc.load_gather(
                                sv, [rows,
                                     jnp.full((16,), src_p_col + h * j_proj + j,
                                              jnp.int32)])
                            plsc.store_scatter(
                                ov, [rows,
                                     jnp.full((16,), h_heads + h * j_proj + j,
                                              jnp.int32)], w * p)
            pltpu.sync_copy(ov, acc.at[didx], add=True)
            return carry

        lax.fori_loop(0, n_chunks, chunk, 0)
        plsc.subcore_barrier()
        pltpu.sync_copy(acc.at[pl.ds(sid * rows_per_sub, rows_per_sub)],
                        out.at[cid].at[pl.ds(sid * rows_per_sub, rows_per_sub)])

    return k


def _pad_edges(src, dst, pad_dst):
    e = src.shape[0]
    e_pad = _round_up(e, _ECHUNK)
    src_p = jnp.pad(src, (0, e_pad - e))
    dst_p = jnp.pad(dst, (0, e_pad - e), constant_values=pad_dst)
    return src_p, dst_p, e_pad


def _combine_l1(accs, bias, h_heads, j_proj, sel_b, sel_s):
    n = accs[0].shape[1]
    c = accs[0].shape[2]
    R = 512
    na = len(accs)

    def body(*refs):
        a_refs = refs[:na]
        b_ref, selb_ref, sels_ref, o_ref = refs[na:]
        v = None
        for a_ref in a_refs:
            a = a_ref[0] + a_ref[1]
            inv = 1.0 / (a + _EPS)
            t = a * jnp.dot(inv, selb_ref[...],
                            preferred_element_type=jnp.float32)
            vj = jnp.dot(t, sels_ref[...], preferred_element_type=jnp.float32)
            v = vj if v is None else v + vj
        o_ref[...] = v + b_ref[...]

    in_specs = (
        [pl.BlockSpec((2, R, c), lambda i: (0, i, 0)) for _ in accs]
        + [pl.BlockSpec((1, j_proj), lambda i: (0, 0)),
           pl.BlockSpec((c, c), lambda i: (0, 0)),
           pl.BlockSpec((c, j_proj), lambda i: (0, 0))]
    )
    return pl.pallas_call(
        body,
        grid=(n // R,),
        in_specs=in_specs,
        out_specs=pl.BlockSpec((R, j_proj), lambda i: (i, 0)),
        out_shape=jax.ShapeDtypeStruct((n, j_proj), jnp.float32),
    )(*accs, bias, sel_b, sel_s)


def _combine_final(acc_a, acc_b, c0):
    n = acc_a.shape[1]
    R = 1024

    def body(a_ref, b_ref, c_ref, o_ref):
        a = a_ref[0] + a_ref[1]
        b = b_ref[0] + b_ref[1]
        r = a[:, 1] / (a[:, 0] + _EPS) + b[:, 1] / (b[:, 0] + _EPS)
        o_ref[...] = (r + c_ref[0]).reshape(R // 128, 128)

    return pl.pallas_call(
        body,
        grid=(n // R,),
        in_specs=[
            pl.BlockSpec((2, R, 2), lambda i: (0, i, 0)),
            pl.BlockSpec((2, R, 2), lambda i: (0, i, 0)),
            pl.BlockSpec(memory_space=pltpu.SMEM),
        ],
        out_specs=pl.BlockSpec((R // 128, 128), lambda i: (i, 0)),
        out_shape=jax.ShapeDtypeStruct((n // 128, 128), jnp.float32),
    )(acc_a, acc_b, c0)


def kernel(sentence_feat, word_feat, W_sent, b_sent, W_word, b_word,
           W_ss1, a_src_ss1, a_dst_ss1, b_ss1,
           W_sw1, a_src_sw1, a_dst_sw1, b_sw1,
           W_ws1, a_src_ws1, a_dst_ws1, b_ws1,
           W_ss2, a_src_ss2, a_dst_ss2, b_ss2,
           W_sw2, a_src_sw2, a_dst_sw2, b_sw2,
           W_ws2, a_src_ws2, a_dst_ws2, b_ws2,
           ei_ss_src, ei_ss_dst, ei_sw_src, ei_sw_dst, ei_ws_src, ei_ws_dst):
    f32 = jnp.float32

    V_sent = jnp.stack([W_ss2 @ a_src_ss2[0], W_ss2 @ a_dst_ss2[0],
                        W_ss2.mean(axis=1), W_ws2 @ a_dst_ws2[0]], axis=1)
    V_word = jnp.stack([W_ws2 @ a_src_ws2[0], W_ws2.mean(axis=1)], axis=1)

    def fold(W1, a_s, a_d, V):
        W1r = W1.reshape(_DIN, _H1, _C1)
        A_s = jnp.einsum('dhc,hc->dh', W1r, a_s)
        A_d = jnp.einsum('dhc,hc->dh', W1r, a_d)
        M = jnp.einsum('dhc,hcj->dhj', W1r,
                       V.reshape(_H1, _C1, V.shape[1]))
        return A_s, A_d, M.reshape(_DIN, _H1 * V.shape[1])

    As_ss, Ad_ss, M_ss = fold(W_ss1, a_src_ss1, a_dst_ss1, V_sent)
    As_sw, Ad_sw, M_sw = fold(W_sw1, a_src_sw1, a_dst_sw1, V_word)
    As_ws, Ad_ws, M_ws = fold(W_ws1, a_src_ws1, a_dst_ws1, V_sent)

    G_src_ss = jnp.concatenate([As_ss, M_ss], axis=1)
    G_src_sw = jnp.concatenate([As_sw, M_sw], axis=1)
    G_src_ws = jnp.concatenate([As_ws, M_ws], axis=1)

    bias_sent = ((b_ss1 + b_ws1) @ V_sent).reshape(1, 4)
    bias_word = (b_sw1 @ V_word).reshape(1, 2)
    c0 = (b_ss2.mean() + b_ws2.mean()).reshape(1)

    bs = b_sent.reshape(1, _DIN)
    bw = b_word.reshape(1, _DIN)

    T_src_ss, T_dst_ss, T_src_sw, T_dst_ws = _dense_fields(
        sentence_feat, W_sent.T, bs, [G_src_ss, Ad_ss, G_src_sw, Ad_ws],
        _NSPAD)
    T_dst_sw, T_src_ws = _dense_fields(word_feat, W_word.T, bw,
                                       [Ad_sw, G_src_ws], _NWPAD)

    ss_s, ss_d, e_ss = _pad_edges(ei_ss_src, ei_ss_dst, _NS)
    sw_s, sw_d, e_sw = _pad_edges(ei_sw_src, ei_sw_dst, _NW)
    ws_s, ws_d, e_ws = _pad_edges(ei_ws_src, ei_ws_dst, _NS)

    z_ns20 = jnp.zeros((_NSPAD // _NSUB, 20), f32)
    z_nw12 = jnp.zeros((_NWPAD // _NSUB, 12), f32)
    z_ns2 = jnp.zeros((_NSPAD // _NSUB, 2), f32)

    acc_ss = _make_edge_gat(e_ss, _NSPAD, 20, 4, _H1, 4, 0, 4, 0)(
        T_src_ss, T_dst_ss, ss_s, ss_d, z_ns20)
    acc_sw = _make_edge_gat(e_sw, _NWPAD, 12, 4, _H1, 2, 0, 4, 0)(
        T_src_sw, T_dst_sw, sw_s, sw_d, z_nw12)
    acc_ws = _make_edge_gat(e_ws, _NSPAD, 20, 4, _H1, 4, 0, 4, 0)(
        T_src_ws, T_dst_ws, ws_s, ws_d, z_ns20)

    def sel_mats(h_heads, j_proj):
        c = h_heads * (1 + j_proj)
        sb = jnp.zeros((c, c), f32)
        ss = jnp.zeros((c, j_proj), f32)
        for h in range(h_heads):
            for j in range(j_proj):
                sb = sb.at[h, h_heads + h * j_proj + j].set(1.0)
                ss = ss.at[h_heads + h * j_proj + j, j].set(1.0)
        return sb, ss

    sb4, ss4 = sel_mats(_H1, 4)
    sb2, ss2m = sel_mats(_H1, 2)
    proj_sent = _combine_l1([acc_ss, acc_ws], bias_sent, _H1, 4, sb4, ss4)
    proj_word = _combine_l1([acc_sw], bias_word, _H1, 2, sb2, ss2m)

    acc_ss2 = _make_edge_gat(e_ss, _NSPAD, 4, 4, 1, 1, 0, 2, 1)(
        proj_sent, proj_sent, ss_s, ss_d, z_ns2)
    acc_ws2 = _make_edge_gat(e_ws, _NSPAD, 2, 4, 1, 1, 0, 1, 3)(
        proj_word, proj_sent, ws_s, ws_d, z_ns2)

    out = _combine_final(acc_ss2, acc_ws2, c0)
    return out.reshape(_NSPAD)[:_NS]
